# Initial kernel scaffold; baseline (speedup 1.0000x reference)
import jax
import jax.numpy as jnp
from jax.experimental import pallas as pl


def kernel(x, edge_index, W, b, gamma, beta):
    raise NotImplementedError("write your pallas kernel here")


# speedup vs baseline: 23.6602x; 23.6602x over previous
import functools

import jax
import jax.numpy as jnp
from jax import lax
from jax.experimental import pallas as pl
from jax.experimental.pallas import tpu as pltpu
from jax.experimental.pallas import tpu_sc as plsc

N = 10000
D = 128
HD = D // 2
E = 320000

NC = 2
NS = 16
NW = NC * NS
CHUNK = 128
J = -(-E // (NW * CHUNK))
E_PAD = NW * J * CHUNK
J2 = E_PAD // (NS * CHUNK)
SPAN = 640
N_ACC = NS * SPAN

_mesh = plsc.VectorSubcoreMesh(core_axis_name="c", subcore_axis_name="s")


@functools.partial(
    pl.kernel,
    out_type=jax.ShapeDtypeStruct((NC * N_ACC,), jnp.float32),
    mesh=_mesh,
    scratch_types=[
        pltpu.VMEM((J, CHUNK), jnp.int32),
        pltpu.VMEM((CHUNK,), jnp.float32),
        pltpu.VMEM((SPAN,), jnp.float32),
        pltpu.VMEM_SHARED((N_ACC,), jnp.float32),
    ],
)
def _sc_degree(col_hbm, out_hbm, col_v, ones_v, zbuf_v, hist_sh):
    cid = lax.axis_index("c")
    sid = lax.axis_index("s")
    wid = sid * NC + cid

    def _fill(i, _):
        zbuf_v[pl.ds(i * 16, 16)] = jnp.zeros((16,), jnp.float32)
        return 0

    lax.fori_loop(0, SPAN // 16, _fill, 0)
    for i in range(CHUNK // 16):
        ones_v[pl.ds(i * 16, 16)] = jnp.ones((16,), jnp.float32)

    pltpu.sync_copy(zbuf_v, hist_sh.at[pl.ds(sid * SPAN, SPAN)])
    plsc.subcore_barrier()

    pltpu.sync_copy(col_hbm.at[wid], col_v)

    def _scat(j, _):
        pltpu.sync_copy(ones_v, hist_sh.at[col_v.at[j]], add=True)
        return 0

    lax.fori_loop(0, J, _scat, 0)
    plsc.subcore_barrier()

    pltpu.sync_copy(hist_sh.at[pl.ds(sid * SPAN, SPAN)], zbuf_v)
    pltpu.sync_copy(zbuf_v, out_hbm.at[pl.ds(cid * N_ACC + sid * SPAN, SPAN)])


@functools.partial(
    pl.kernel,
    out_type=jax.ShapeDtypeStruct((NC, N_ACC, HD), jnp.float32),
    mesh=_mesh,
    compiler_params=pltpu.CompilerParams(use_tc_tiling_on_sc=False),
    scratch_types=[
        pltpu.VMEM((J2, CHUNK), jnp.int32),
        pltpu.VMEM((J2, CHUNK), jnp.int32),
        pltpu.VMEM((CHUNK, HD), jnp.float32),
        pltpu.VMEM((CHUNK, HD), jnp.float32),
        pltpu.SemaphoreType.DMA,
        pltpu.SemaphoreType.DMA,
        pltpu.VMEM_SHARED((N_ACC, HD), jnp.float32),
    ],
)
def _sc_aggregate(g_hbm, row_hbm, col_hbm, out_hbm,
                  row_v, col_v, buf_a, buf_b, sem_a, sem_b, agg_sh):
    cid = lax.axis_index("c")
    sid = lax.axis_index("s")
    base = sid * SPAN

    def _zrow(r, _):
        for i in range(HD // 16):
            buf_a[r, pl.ds(i * 16, 16)] = jnp.zeros((16,), jnp.float32)
        return 0

    lax.fori_loop(0, CHUNK, _zrow, 0)
    for k in range(SPAN // CHUNK):
        pltpu.sync_copy(buf_a, agg_sh.at[pl.ds(base + k * CHUNK, CHUNK), :])
    plsc.subcore_barrier()

    pltpu.sync_copy(row_hbm.at[cid * NS + sid], row_v)
    pltpu.sync_copy(col_hbm.at[sid], col_v)

    def _step(j, _):
        pltpu.sync_copy(g_hbm.at[row_v.at[j]], buf_a)
        pltpu.sync_copy(buf_a, agg_sh.at[col_v.at[j]], add=True)
        return 0

    lax.fori_loop(0, J2, _step, 0)

    plsc.subcore_barrier()
    for k in range(SPAN // CHUNK):
        pltpu.sync_copy(agg_sh.at[pl.ds(base + k * CHUNK, CHUNK), :], buf_a)
        pltpu.sync_copy(buf_a, out_hbm.at[cid, pl.ds(base + k * CHUNK, CHUNK), :])


def _tc_transform_body(x_ref, w_ref, h0_ref, h1_ref, g2_ref, dis_ref):
    deg = h0_ref[...] + h1_ref[...] + 1.0
    dis = lax.rsqrt(deg)
    h = jnp.dot(x_ref[...], w_ref[...], preferred_element_type=jnp.float32)
    g = h * dis
    g2_ref[pl.ds(0, N), :] = g[:, :HD]
    g2_ref[pl.ds(N, N), :] = g[:, HD:]
    dis_ref[...] = dis


def _tc_bn_body(a0_ref, a1_ref, gl_ref, gr_ref, dis_ref, b_ref, gam_ref,
                bet_ref, out_ref):
    dis = dis_ref[...]
    agg = jnp.concatenate([a0_ref[...] + gl_ref[...],
                           a1_ref[...] + gr_ref[...]], axis=1)
    pre = dis * agg + b_ref[...]
    mean = jnp.mean(pre, axis=0, keepdims=True)
    d = pre - mean
    var = jnp.mean(d * d, axis=0, keepdims=True)
    out = d * lax.rsqrt(var + 1e-5) * gam_ref[...] + bet_ref[...]
    out_ref[...] = jnp.maximum(out, 0.0)


def kernel(x, edge_index, W, b, gamma, beta):
    row = edge_index[0]
    col = edge_index[1]
    pad = E_PAD - E
    pad_row = (jnp.arange(pad, dtype=jnp.int32) % 128)
    pad_col = N + (jnp.arange(pad, dtype=jnp.int32) % 16)
    row_pad = jnp.concatenate([row, pad_row])
    col_pad = jnp.concatenate([col, pad_col])
    col_p1 = col_pad.reshape(NW, J, CHUNK)
    col_p2 = col_pad.reshape(NS, J2, CHUNK)
    row_p2 = jnp.stack([row_pad, row_pad + N]).reshape(NC * NS, J2, CHUNK)

    hist = _sc_degree(col_p1)
    h0 = hist[:N].reshape(N, 1)
    h1 = hist[N_ACC:N_ACC + N].reshape(N, 1)

    g2, dis = pl.pallas_call(
        _tc_transform_body,
        out_shape=(
            jax.ShapeDtypeStruct((2 * N, HD), jnp.float32),
            jax.ShapeDtypeStruct((N, 1), jnp.float32),
        ),
    )(x, W, h0, h1)

    agg = _sc_aggregate(g2, row_p2, col_p2)
    a0 = agg[0, :N]
    a1 = agg[1, :N]

    out = pl.pallas_call(
        _tc_bn_body,
        out_shape=jax.ShapeDtypeStruct((N, D), jnp.float32),
    )(a0, a1, g2[:N], g2[N:], dis, b.reshape(1, D), gamma.reshape(1, D),
      beta.reshape(1, D))
    return out


# speedup vs baseline: 28.2136x; 1.1924x over previous
import functools

import jax
import jax.numpy as jnp
from jax import lax
from jax.experimental import pallas as pl
from jax.experimental.pallas import tpu as pltpu
from jax.experimental.pallas import tpu_sc as plsc

N = 10000
D = 128
HD = D // 2
E = 320000

NC = 2
NS = 16
NW = NC * NS
CHUNK = 128
J = -(-E // (NW * CHUNK))
E_PAD = NW * J * CHUNK
J2 = E_PAD // (NS * CHUNK)
KB = 2
SPAN = 640
N_ACC = NS * SPAN

_mesh = plsc.VectorSubcoreMesh(core_axis_name="c", subcore_axis_name="s")


@functools.partial(
    pl.kernel,
    out_type=jax.ShapeDtypeStruct((NC * N_ACC,), jnp.float32),
    mesh=_mesh,
    scratch_types=[
        pltpu.VMEM((J, CHUNK), jnp.int32),
        pltpu.VMEM((CHUNK,), jnp.float32),
        pltpu.VMEM((SPAN,), jnp.float32),
        pltpu.VMEM_SHARED((N_ACC,), jnp.float32),
    ],
)
def _sc_degree(col_hbm, out_hbm, col_v, ones_v, zbuf_v, hist_sh):
    cid = lax.axis_index("c")
    sid = lax.axis_index("s")
    wid = sid * NC + cid

    def _fill(i, _):
        zbuf_v[pl.ds(i * 16, 16)] = jnp.zeros((16,), jnp.float32)
        return 0

    lax.fori_loop(0, SPAN // 16, _fill, 0)
    for i in range(CHUNK // 16):
        ones_v[pl.ds(i * 16, 16)] = jnp.ones((16,), jnp.float32)

    pltpu.sync_copy(zbuf_v, hist_sh.at[pl.ds(sid * SPAN, SPAN)])
    plsc.subcore_barrier()

    pltpu.sync_copy(col_hbm.at[wid], col_v)

    def _scat(j, _):
        pltpu.sync_copy(ones_v, hist_sh.at[col_v.at[j]], add=True)
        return 0

    lax.fori_loop(0, J, _scat, 0)
    plsc.subcore_barrier()

    pltpu.sync_copy(hist_sh.at[pl.ds(sid * SPAN, SPAN)], zbuf_v)
    pltpu.sync_copy(zbuf_v, out_hbm.at[pl.ds(cid * N_ACC + sid * SPAN, SPAN)])


@functools.partial(
    pl.kernel,
    out_type=jax.ShapeDtypeStruct((NC, N_ACC, HD), jnp.float32),
    mesh=_mesh,
    compiler_params=pltpu.CompilerParams(use_tc_tiling_on_sc=False),
    scratch_types=[
        pltpu.VMEM((J2 // KB, KB * CHUNK), jnp.int32),
        pltpu.VMEM((J2 // KB, KB * CHUNK), jnp.int32),
        pltpu.VMEM((KB * CHUNK, HD), jnp.float32),
        pltpu.VMEM((KB * CHUNK, HD), jnp.float32),
        pltpu.SemaphoreType.DMA,
        pltpu.SemaphoreType.DMA,
        pltpu.VMEM_SHARED((N_ACC, HD), jnp.float32),
    ],
)
def _sc_aggregate(g_hbm, row_hbm, col_hbm, out_hbm,
                  row_v, col_v, buf_a, buf_b, sem_a, sem_b, agg_sh):
    cid = lax.axis_index("c")
    sid = lax.axis_index("s")
    base = sid * SPAN

    def _zrow(r, _):
        for i in range(HD // 16):
            buf_a[r, pl.ds(i * 16, 16)] = jnp.zeros((16,), jnp.float32)
        return 0

    lax.fori_loop(0, CHUNK, _zrow, 0)
    for k in range(SPAN // CHUNK):
        pltpu.sync_copy(buf_a.at[pl.ds(0, CHUNK), :],
                        agg_sh.at[pl.ds(base + k * CHUNK, CHUNK), :])
    plsc.subcore_barrier()

    pltpu.sync_copy(row_hbm.at[cid * NS + sid], row_v)
    pltpu.sync_copy(col_hbm.at[sid], col_v)

    def _step(j, _):
        pltpu.sync_copy(g_hbm.at[row_v.at[j]], buf_a)
        pltpu.sync_copy(buf_a, agg_sh.at[col_v.at[j]], add=True)
        return 0

    lax.fori_loop(0, J2 // KB, _step, 0)

    plsc.subcore_barrier()
    for k in range(SPAN // CHUNK):
        pltpu.sync_copy(agg_sh.at[pl.ds(base + k * CHUNK, CHUNK), :],
                        buf_a.at[pl.ds(0, CHUNK), :])
        pltpu.sync_copy(buf_a.at[pl.ds(0, CHUNK), :],
                        out_hbm.at[cid, pl.ds(base + k * CHUNK, CHUNK), :])


def _tc_transform_body(x_ref, w_ref, h0_ref, h1_ref, g2_ref, dis_ref):
    deg = h0_ref[...] + h1_ref[...] + 1.0
    dis = lax.rsqrt(deg)
    h = jnp.dot(x_ref[...], w_ref[...], preferred_element_type=jnp.float32)
    g = h * dis
    g2_ref[pl.ds(0, N), :] = g[:, :HD]
    g2_ref[pl.ds(N, N), :] = g[:, HD:]
    dis_ref[...] = dis


def _tc_bn_body(a0_ref, a1_ref, gl_ref, gr_ref, dis_ref, b_ref, gam_ref,
                bet_ref, out_ref):
    dis = dis_ref[...]
    agg = jnp.concatenate([a0_ref[...] + gl_ref[...],
                           a1_ref[...] + gr_ref[...]], axis=1)
    pre = dis * agg + b_ref[...]
    mean = jnp.mean(pre, axis=0, keepdims=True)
    d = pre - mean
    var = jnp.mean(d * d, axis=0, keepdims=True)
    out = d * lax.rsqrt(var + 1e-5) * gam_ref[...] + bet_ref[...]
    out_ref[...] = jnp.maximum(out, 0.0)


def kernel(x, edge_index, W, b, gamma, beta):
    row = edge_index[0]
    col = edge_index[1]
    pad = E_PAD - E
    pad_row = (jnp.arange(pad, dtype=jnp.int32) % 128)
    pad_col = N + (jnp.arange(pad, dtype=jnp.int32) % 16)
    row_pad = jnp.concatenate([row, pad_row])
    col_pad = jnp.concatenate([col, pad_col])
    col_p1 = col_pad.reshape(NW, J, CHUNK)
    col_p2 = col_pad.reshape(NS, J2 // KB, KB * CHUNK)
    row_p2 = jnp.stack([row_pad, row_pad + N]).reshape(
        NC * NS, J2 // KB, KB * CHUNK)

    hist = _sc_degree(col_p1)
    h0 = hist[:N].reshape(N, 1)
    h1 = hist[N_ACC:N_ACC + N].reshape(N, 1)

    g2, dis = pl.pallas_call(
        _tc_transform_body,
        out_shape=(
            jax.ShapeDtypeStruct((2 * N, HD), jnp.float32),
            jax.ShapeDtypeStruct((N, 1), jnp.float32),
        ),
    )(x, W, h0, h1)

    agg = _sc_aggregate(g2, row_p2, col_p2)
    a0 = agg[0, :N]
    a1 = agg[1, :N]

    out = pl.pallas_call(
        _tc_bn_body,
        out_shape=jax.ShapeDtypeStruct((N, D), jnp.float32),
    )(a0, a1, g2[:N], g2[N:], dis, b.reshape(1, D), gamma.reshape(1, D),
      beta.reshape(1, D))
    return out


# speedup vs baseline: 37.0592x; 1.3135x over previous
import functools

import jax
import jax.numpy as jnp
from jax import lax
from jax.experimental import pallas as pl
from jax.experimental.pallas import tpu as pltpu
from jax.experimental.pallas import tpu_sc as plsc

N = 10000
D = 128
HD = D // 2
E = 320000

NC = 2
NS = 16
NW = NC * NS
CHUNK = 128
J = -(-E // (NW * CHUNK))
E_PAD = NW * J * CHUNK
J2 = E_PAD // (NS * CHUNK)
KB = 2
NSLAB = J2 // KB
SPAN = 640
N_ACC = NS * SPAN

_mesh = plsc.VectorSubcoreMesh(core_axis_name="c", subcore_axis_name="s")


@functools.partial(
    pl.kernel,
    out_type=jax.ShapeDtypeStruct((NC * N_ACC,), jnp.float32),
    mesh=_mesh,
    scratch_types=[
        pltpu.VMEM((J, CHUNK), jnp.int32),
        pltpu.VMEM((CHUNK,), jnp.float32),
        pltpu.VMEM((SPAN,), jnp.float32),
        pltpu.VMEM_SHARED((N_ACC,), jnp.float32),
    ],
)
def _sc_degree(col_hbm, out_hbm, col_v, ones_v, zbuf_v, hist_sh):
    cid = lax.axis_index("c")
    sid = lax.axis_index("s")
    wid = sid * NC + cid

    def _fill(i, _):
        zbuf_v[pl.ds(i * 16, 16)] = jnp.zeros((16,), jnp.float32)
        return 0

    lax.fori_loop(0, SPAN // 16, _fill, 0)
    for i in range(CHUNK // 16):
        ones_v[pl.ds(i * 16, 16)] = jnp.ones((16,), jnp.float32)

    pltpu.sync_copy(zbuf_v, hist_sh.at[pl.ds(sid * SPAN, SPAN)])
    plsc.subcore_barrier()

    pltpu.sync_copy(col_hbm.at[wid], col_v)

    def _scat(j, _):
        pltpu.sync_copy(ones_v, hist_sh.at[col_v.at[j]], add=True)
        return 0

    lax.fori_loop(0, J, _scat, 0)
    plsc.subcore_barrier()

    pltpu.sync_copy(hist_sh.at[pl.ds(sid * SPAN, SPAN)], zbuf_v)
    pltpu.sync_copy(zbuf_v, out_hbm.at[pl.ds(cid * N_ACC + sid * SPAN, SPAN)])


@functools.partial(
    pl.kernel,
    out_type=jax.ShapeDtypeStruct((NC, N_ACC, HD), jnp.float32),
    mesh=_mesh,
    compiler_params=pltpu.CompilerParams(use_tc_tiling_on_sc=False),
    scratch_types=[
        pltpu.VMEM((J2 // KB, KB * CHUNK), jnp.int32),
        pltpu.VMEM((J2 // KB, KB * CHUNK), jnp.int32),
        pltpu.VMEM((KB * CHUNK, HD), jnp.float32),
        pltpu.VMEM((KB * CHUNK, HD), jnp.float32),
        pltpu.SemaphoreType.DMA,
        pltpu.SemaphoreType.DMA,
        pltpu.VMEM_SHARED((N_ACC, HD), jnp.float32),
    ],
)
def _sc_aggregate(g_hbm, row_hbm, col_hbm, out_hbm,
                  row_v, col_v, buf_a, buf_b, sem_a, sem_b, agg_sh):
    cid = lax.axis_index("c")
    sid = lax.axis_index("s")
    base = sid * SPAN

    def _zrow(r, _):
        for i in range(HD // 16):
            buf_a[r, pl.ds(i * 16, 16)] = jnp.zeros((16,), jnp.float32)
        return 0

    lax.fori_loop(0, CHUNK, _zrow, 0)
    for k in range(SPAN // CHUNK):
        pltpu.sync_copy(buf_a.at[pl.ds(0, CHUNK), :],
                        agg_sh.at[pl.ds(base + k * CHUNK, CHUNK), :])
    plsc.subcore_barrier()

    pltpu.sync_copy(row_hbm.at[cid * NS + sid], row_v)
    pltpu.sync_copy(col_hbm.at[sid], col_v)

    pltpu.async_copy(g_hbm.at[row_v.at[0]], buf_a, sem_a)

    def _step(jj, _):
        j = jj * 2
        pltpu.async_copy(g_hbm.at[row_v.at[j + 1]], buf_b, sem_b)
        pltpu.make_async_copy(g_hbm.at[row_v.at[0]], buf_a, sem_a).wait()
        pltpu.sync_copy(buf_a, agg_sh.at[col_v.at[j]], add=True)
        pltpu.async_copy(g_hbm.at[row_v.at[j + 2]], buf_a, sem_a)
        pltpu.make_async_copy(g_hbm.at[row_v.at[0]], buf_b, sem_b).wait()
        pltpu.sync_copy(buf_b, agg_sh.at[col_v.at[j + 1]], add=True)
        return 0

    lax.fori_loop(0, (NSLAB - 1) // 2, _step, 0)
    pltpu.make_async_copy(g_hbm.at[row_v.at[0]], buf_a, sem_a).wait()
    pltpu.sync_copy(buf_a, agg_sh.at[col_v.at[NSLAB - 1]], add=True)

    plsc.subcore_barrier()
    for k in range(SPAN // CHUNK):
        pltpu.sync_copy(agg_sh.at[pl.ds(base + k * CHUNK, CHUNK), :],
                        buf_a.at[pl.ds(0, CHUNK), :])
        pltpu.sync_copy(buf_a.at[pl.ds(0, CHUNK), :],
                        out_hbm.at[cid, pl.ds(base + k * CHUNK, CHUNK), :])


def _tc_transform_body(x_ref, w_ref, h0_ref, h1_ref, g2_ref, dis_ref):
    deg = h0_ref[...] + h1_ref[...] + 1.0
    dis = lax.rsqrt(deg)
    h = jnp.dot(x_ref[...], w_ref[...], preferred_element_type=jnp.float32)
    g = h * dis
    g2_ref[pl.ds(0, N), :] = g[:, :HD]
    g2_ref[pl.ds(N, N), :] = g[:, HD:]
    dis_ref[...] = dis


def _tc_bn_body(a0_ref, a1_ref, gl_ref, gr_ref, dis_ref, b_ref, gam_ref,
                bet_ref, out_ref):
    dis = dis_ref[...]
    agg = jnp.concatenate([a0_ref[...] + gl_ref[...],
                           a1_ref[...] + gr_ref[...]], axis=1)
    pre = dis * agg + b_ref[...]
    mean = jnp.mean(pre, axis=0, keepdims=True)
    d = pre - mean
    var = jnp.mean(d * d, axis=0, keepdims=True)
    out = d * lax.rsqrt(var + 1e-5) * gam_ref[...] + bet_ref[...]
    out_ref[...] = jnp.maximum(out, 0.0)


def kernel(x, edge_index, W, b, gamma, beta):
    row = edge_index[0]
    col = edge_index[1]
    pad = E_PAD - E
    pad_row = (jnp.arange(pad, dtype=jnp.int32) % 128)
    pad_col = N + (jnp.arange(pad, dtype=jnp.int32) % 16)
    row_pad = jnp.concatenate([row, pad_row])
    col_pad = jnp.concatenate([col, pad_col])
    col_p1 = col_pad.reshape(NW, J, CHUNK)
    col_p2 = col_pad.reshape(NS, J2 // KB, KB * CHUNK)
    row_p2 = jnp.stack([row_pad, row_pad + N]).reshape(
        NC * NS, J2 // KB, KB * CHUNK)

    hist = _sc_degree(col_p1)
    h0 = hist[:N].reshape(N, 1)
    h1 = hist[N_ACC:N_ACC + N].reshape(N, 1)

    g2, dis = pl.pallas_call(
        _tc_transform_body,
        out_shape=(
            jax.ShapeDtypeStruct((2 * N, HD), jnp.float32),
            jax.ShapeDtypeStruct((N, 1), jnp.float32),
        ),
    )(x, W, h0, h1)

    agg = _sc_aggregate(g2, row_p2, col_p2)
    a0 = agg[0, :N]
    a1 = agg[1, :N]

    out = pl.pallas_call(
        _tc_bn_body,
        out_shape=jax.ShapeDtypeStruct((N, D), jnp.float32),
    )(a0, a1, g2[:N], g2[N:], dis, b.reshape(1, D), gamma.reshape(1, D),
      beta.reshape(1, D))
    return out


# speedup vs baseline: 38.7952x; 1.0468x over previous
import functools

import jax
import jax.numpy as jnp
from jax import lax
from jax.experimental import pallas as pl
from jax.experimental.pallas import tpu as pltpu
from jax.experimental.pallas import tpu_sc as plsc

N = 10000
D = 128
HD = D // 2
E = 320000

NC = 2
NS = 16
NW = NC * NS
CHUNK = 128
J = 80
E_PAD = NW * J * CHUNK
J2 = E_PAD // (NS * CHUNK)
KB = 1
NSLAB = J2 // KB
NBUF = 4
SPAN = 640
N_ACC = NS * SPAN

_mesh = plsc.VectorSubcoreMesh(core_axis_name="c", subcore_axis_name="s")


@functools.partial(
    pl.kernel,
    out_type=jax.ShapeDtypeStruct((NC * N_ACC,), jnp.float32),
    mesh=_mesh,
    scratch_types=[
        pltpu.VMEM((J, CHUNK), jnp.int32),
        pltpu.VMEM((CHUNK,), jnp.float32),
        pltpu.VMEM((SPAN,), jnp.float32),
        pltpu.VMEM_SHARED((N_ACC,), jnp.float32),
    ],
)
def _sc_degree(col_hbm, out_hbm, col_v, ones_v, zbuf_v, hist_sh):
    cid = lax.axis_index("c")
    sid = lax.axis_index("s")
    wid = sid * NC + cid

    def _fill(i, _):
        zbuf_v[pl.ds(i * 16, 16)] = jnp.zeros((16,), jnp.float32)
        return 0

    lax.fori_loop(0, SPAN // 16, _fill, 0)
    for i in range(CHUNK // 16):
        ones_v[pl.ds(i * 16, 16)] = jnp.ones((16,), jnp.float32)

    pltpu.sync_copy(zbuf_v, hist_sh.at[pl.ds(sid * SPAN, SPAN)])
    plsc.subcore_barrier()

    pltpu.sync_copy(col_hbm.at[wid], col_v)

    def _scat(j, _):
        pltpu.sync_copy(ones_v, hist_sh.at[col_v.at[j]], add=True)
        return 0

    lax.fori_loop(0, J, _scat, 0)
    plsc.subcore_barrier()

    pltpu.sync_copy(hist_sh.at[pl.ds(sid * SPAN, SPAN)], zbuf_v)
    pltpu.sync_copy(zbuf_v, out_hbm.at[pl.ds(cid * N_ACC + sid * SPAN, SPAN)])


@functools.partial(
    pl.kernel,
    out_type=jax.ShapeDtypeStruct((NC, N_ACC, HD), jnp.float32),
    mesh=_mesh,
    compiler_params=pltpu.CompilerParams(use_tc_tiling_on_sc=False),
    scratch_types=[
        pltpu.VMEM((NSLAB, KB * CHUNK), jnp.int32),
        pltpu.VMEM((NSLAB, KB * CHUNK), jnp.int32),
        [pltpu.VMEM((KB * CHUNK, HD), jnp.float32) for _ in range(NBUF)],
        [pltpu.SemaphoreType.DMA for _ in range(NBUF)],
        pltpu.VMEM_SHARED((N_ACC, HD), jnp.float32),
    ],
)
def _sc_aggregate(g_hbm, row_hbm, col_hbm, out_hbm,
                  row_v, col_v, bufs, sems, agg_sh):
    cid = lax.axis_index("c")
    sid = lax.axis_index("s")
    base = sid * SPAN

    def _zrow(r, _):
        for i in range(HD // 16):
            bufs[0][r, pl.ds(i * 16, 16)] = jnp.zeros((16,), jnp.float32)
        return 0

    lax.fori_loop(0, CHUNK, _zrow, 0)
    for k in range(SPAN // CHUNK):
        pltpu.sync_copy(bufs[0].at[pl.ds(0, CHUNK), :],
                        agg_sh.at[pl.ds(base + k * CHUNK, CHUNK), :])
    plsc.subcore_barrier()

    pltpu.sync_copy(row_hbm.at[cid * NS + sid], row_v)
    pltpu.sync_copy(col_hbm.at[sid], col_v)

    for t in range(NBUF):
        pltpu.async_copy(g_hbm.at[row_v.at[t]], bufs[t], sems[t])

    def _step(i, _):
        for t in range(NBUF):
            j = i * NBUF + t
            pltpu.make_async_copy(g_hbm.at[row_v.at[0]], bufs[t],
                                  sems[t]).wait()
            pltpu.sync_copy(bufs[t], agg_sh.at[col_v.at[j]], add=True)

            @pl.when(j + NBUF < NSLAB)
            def _():
                pltpu.async_copy(g_hbm.at[row_v.at[j + NBUF]], bufs[t],
                                 sems[t])
        return 0

    lax.fori_loop(0, NSLAB // NBUF, _step, 0)

    plsc.subcore_barrier()
    for k in range(SPAN // CHUNK):
        t = k % NBUF
        pltpu.sync_copy(agg_sh.at[pl.ds(base + k * CHUNK, CHUNK), :],
                        bufs[t].at[pl.ds(0, CHUNK), :])
        pltpu.sync_copy(bufs[t].at[pl.ds(0, CHUNK), :],
                        out_hbm.at[cid, pl.ds(base + k * CHUNK, CHUNK), :])


def _tc_transform_body(x_ref, w_ref, h0_ref, h1_ref, g2_ref, dis_ref):
    deg = h0_ref[...] + h1_ref[...] + 1.0
    dis = lax.rsqrt(deg)
    h = jnp.dot(x_ref[...], w_ref[...], preferred_element_type=jnp.float32)
    g = h * dis
    g2_ref[pl.ds(0, N), :] = g[:, :HD]
    g2_ref[pl.ds(N, N), :] = g[:, HD:]
    dis_ref[...] = dis


def _tc_bn_body(a0_ref, a1_ref, gl_ref, gr_ref, dis_ref, b_ref, gam_ref,
                bet_ref, out_ref):
    dis = dis_ref[...]
    agg = jnp.concatenate([a0_ref[...] + gl_ref[...],
                           a1_ref[...] + gr_ref[...]], axis=1)
    pre = dis * agg + b_ref[...]
    mean = jnp.mean(pre, axis=0, keepdims=True)
    d = pre - mean
    var = jnp.mean(d * d, axis=0, keepdims=True)
    out = d * lax.rsqrt(var + 1e-5) * gam_ref[...] + bet_ref[...]
    out_ref[...] = jnp.maximum(out, 0.0)


def kernel(x, edge_index, W, b, gamma, beta):
    row = edge_index[0]
    col = edge_index[1]
    pad = E_PAD - E
    pad_row = (jnp.arange(pad, dtype=jnp.int32) % 128)
    pad_col = N + (jnp.arange(pad, dtype=jnp.int32) % 16)
    row_pad = jnp.concatenate([row, pad_row])
    col_pad = jnp.concatenate([col, pad_col])
    col_p1 = col_pad.reshape(NW, J, CHUNK)
    col_p2 = col_pad.reshape(NS, NSLAB, KB * CHUNK)
    row_p2 = jnp.stack([row_pad, row_pad + N]).reshape(
        NC * NS, NSLAB, KB * CHUNK)

    hist = _sc_degree(col_p1)
    h0 = hist[:N].reshape(N, 1)
    h1 = hist[N_ACC:N_ACC + N].reshape(N, 1)

    g2, dis = pl.pallas_call(
        _tc_transform_body,
        out_shape=(
            jax.ShapeDtypeStruct((2 * N, HD), jnp.float32),
            jax.ShapeDtypeStruct((N, 1), jnp.float32),
        ),
    )(x, W, h0, h1)

    agg = _sc_aggregate(g2, row_p2, col_p2)
    a0 = agg[0, :N]
    a1 = agg[1, :N]

    out = pl.pallas_call(
        _tc_bn_body,
        out_shape=jax.ShapeDtypeStruct((N, D), jnp.float32),
    )(a0, a1, g2[:N], g2[N:], dis, b.reshape(1, D), gamma.reshape(1, D),
      beta.reshape(1, D))
    return out


# speedup vs baseline: 39.6821x; 1.0229x over previous
import functools

import jax
import jax.numpy as jnp
from jax import lax
from jax.experimental import pallas as pl
from jax.experimental.pallas import tpu as pltpu
from jax.experimental.pallas import tpu_sc as plsc

N = 10000
D = 128
HD = D // 2
E = 320000

NC = 2
NS = 16
NW = NC * NS
CHUNK = 128
J = 80
E_PAD = NW * J * CHUNK
J2 = E_PAD // (NS * CHUNK)
KB = 1
NSLAB = J2 // KB
NBUF = 4
SPAN = 640
N_ACC = NS * SPAN

_mesh = plsc.VectorSubcoreMesh(core_axis_name="c", subcore_axis_name="s")


@functools.partial(
    pl.kernel,
    out_type=jax.ShapeDtypeStruct((NC * N_ACC,), jnp.float32),
    mesh=_mesh,
    scratch_types=[
        pltpu.VMEM((J, CHUNK), jnp.int32),
        pltpu.VMEM((CHUNK,), jnp.float32),
        pltpu.VMEM((SPAN,), jnp.float32),
        pltpu.VMEM_SHARED((N_ACC,), jnp.float32),
    ],
)
def _sc_degree(col_hbm, out_hbm, col_v, ones_v, zbuf_v, hist_sh):
    cid = lax.axis_index("c")
    sid = lax.axis_index("s")
    wid = sid * NC + cid

    def _fill(i, _):
        zbuf_v[pl.ds(i * 16, 16)] = jnp.zeros((16,), jnp.float32)
        return 0

    lax.fori_loop(0, SPAN // 16, _fill, 0)
    for i in range(CHUNK // 16):
        ones_v[pl.ds(i * 16, 16)] = jnp.ones((16,), jnp.float32)

    pltpu.sync_copy(zbuf_v, hist_sh.at[pl.ds(sid * SPAN, SPAN)])
    plsc.subcore_barrier()

    pltpu.sync_copy(col_hbm.at[wid], col_v)

    def _scat(j, _):
        pltpu.sync_copy(ones_v, hist_sh.at[col_v.at[j]], add=True)
        return 0

    lax.fori_loop(0, J, _scat, 0)
    plsc.subcore_barrier()

    pltpu.sync_copy(hist_sh.at[pl.ds(sid * SPAN, SPAN)], zbuf_v)
    pltpu.sync_copy(zbuf_v, out_hbm.at[pl.ds(cid * N_ACC + sid * SPAN, SPAN)])


@functools.partial(
    pl.kernel,
    out_type=jax.ShapeDtypeStruct((NC, N_ACC, HD), jnp.float32),
    mesh=_mesh,
    compiler_params=pltpu.CompilerParams(use_tc_tiling_on_sc=False),
    scratch_types=[
        pltpu.VMEM((NSLAB, KB * CHUNK), jnp.int32),
        pltpu.VMEM((NSLAB, KB * CHUNK), jnp.int32),
        [pltpu.VMEM((KB * CHUNK, HD), jnp.float32) for _ in range(NBUF)],
        [pltpu.SemaphoreType.DMA for _ in range(NBUF)],
        pltpu.VMEM_SHARED((N_ACC, HD), jnp.float32),
    ],
)
def _sc_aggregate(gl_hbm, gr_hbm, row_hbm, col_hbm, out_hbm,
                  row_v, col_v, bufs, sems, agg_sh):
    cid = lax.axis_index("c")
    sid = lax.axis_index("s")
    base = sid * SPAN

    def _zrow(r, _):
        for i in range(HD // 16):
            bufs[0][r, pl.ds(i * 16, 16)] = jnp.zeros((16,), jnp.float32)
        return 0

    lax.fori_loop(0, CHUNK, _zrow, 0)
    for k in range(SPAN // CHUNK):
        pltpu.sync_copy(bufs[0].at[pl.ds(0, CHUNK), :],
                        agg_sh.at[pl.ds(base + k * CHUNK, CHUNK), :])
    plsc.subcore_barrier()

    pltpu.sync_copy(row_hbm.at[sid], row_v)
    pltpu.sync_copy(col_hbm.at[sid], col_v)

    def _run(g_hbm):
        for t in range(NBUF):
            pltpu.async_copy(g_hbm.at[row_v.at[t]], bufs[t], sems[t])

        def _step(i, _):
            for t in range(NBUF):
                j = i * NBUF + t
                pltpu.make_async_copy(g_hbm.at[row_v.at[0]], bufs[t],
                                      sems[t]).wait()
                pltpu.sync_copy(bufs[t], agg_sh.at[col_v.at[j]], add=True)

                @pl.when(j + NBUF < NSLAB)
                def _():
                    pltpu.async_copy(g_hbm.at[row_v.at[j + NBUF]], bufs[t],
                                     sems[t])
            return 0

        lax.fori_loop(0, NSLAB // NBUF, _step, 0)

    @pl.when(cid == 0)
    def _():
        _run(gl_hbm)

    @pl.when(cid == 1)
    def _():
        _run(gr_hbm)

    plsc.subcore_barrier()
    for k in range(SPAN // CHUNK):
        t = k % NBUF
        pltpu.sync_copy(agg_sh.at[pl.ds(base + k * CHUNK, CHUNK), :],
                        bufs[t].at[pl.ds(0, CHUNK), :])
        pltpu.sync_copy(bufs[t].at[pl.ds(0, CHUNK), :],
                        out_hbm.at[cid, pl.ds(base + k * CHUNK, CHUNK), :])


def _tc_matmul_body(x_ref, w_ref, h_ref):
    h_ref[...] = jnp.dot(x_ref[...], w_ref[...],
                         preferred_element_type=jnp.float32)


def _tc_transform_body(h_ref, h0_ref, h1_ref, gl_ref, gr_ref, dis_ref):
    deg = h0_ref[...] + h1_ref[...] + 1.0
    dis = lax.rsqrt(deg)
    g = h_ref[...] * dis
    gl_ref[...] = g[:, :HD]
    gr_ref[...] = g[:, HD:]
    dis_ref[...] = dis


def _tc_bn_body(a0_ref, a1_ref, gl_ref, gr_ref, dis_ref, b_ref, gam_ref,
                bet_ref, out_ref):
    dis = dis_ref[...]
    agg = jnp.concatenate([a0_ref[...] + gl_ref[...],
                           a1_ref[...] + gr_ref[...]], axis=1)
    pre = dis * agg + b_ref[...]
    mean = jnp.mean(pre, axis=0, keepdims=True)
    d = pre - mean
    var = jnp.mean(d * d, axis=0, keepdims=True)
    out = d * lax.rsqrt(var + 1e-5) * gam_ref[...] + bet_ref[...]
    out_ref[...] = jnp.maximum(out, 0.0)


def kernel(x, edge_index, W, b, gamma, beta):
    row = edge_index[0]
    col = edge_index[1]
    pad = E_PAD - E
    pad_row = (jnp.arange(pad, dtype=jnp.int32) % 128)
    pad_col = N + (jnp.arange(pad, dtype=jnp.int32) % 16)
    row_pad = jnp.concatenate([row, pad_row])
    col_pad = jnp.concatenate([col, pad_col])
    col_p1 = col_pad.reshape(NW, J, CHUNK)
    col_p2 = col_pad.reshape(NS, NSLAB, KB * CHUNK)
    row_p2 = row_pad.reshape(NS, NSLAB, KB * CHUNK)

    hist = _sc_degree(col_p1)
    h0 = hist[:N].reshape(N, 1)
    h1 = hist[N_ACC:N_ACC + N].reshape(N, 1)

    h = pl.pallas_call(
        _tc_matmul_body,
        out_shape=jax.ShapeDtypeStruct((N, D), jnp.float32),
    )(x, W)

    gl, gr, dis = pl.pallas_call(
        _tc_transform_body,
        out_shape=(
            jax.ShapeDtypeStruct((N, HD), jnp.float32),
            jax.ShapeDtypeStruct((N, HD), jnp.float32),
            jax.ShapeDtypeStruct((N, 1), jnp.float32),
        ),
    )(h, h0, h1)

    agg = _sc_aggregate(gl, gr, row_p2, col_p2)
    a0 = agg[0, :N]
    a1 = agg[1, :N]

    out = pl.pallas_call(
        _tc_bn_body,
        out_shape=jax.ShapeDtypeStruct((N, D), jnp.float32),
    )(a0, a1, gl, gr, dis, b.reshape(1, D), gamma.reshape(1, D),
      beta.reshape(1, D))
    return out


# speedup vs baseline: 42.0267x; 1.0591x over previous
import functools

import jax
import jax.numpy as jnp
from jax import lax
from jax.experimental import pallas as pl
from jax.experimental.pallas import tpu as pltpu
from jax.experimental.pallas import tpu_sc as plsc

N = 10000
D = 128
HD = D // 2
E = 320000

NC = 2
NS = 16
NW = NC * NS
CHUNK = 128

EW = E // NW
JD = EW // CHUNK
TW = EW - JD * CHUNK

ET = E // NS
JA = ET // CHUNK
TA = ET - JA * CHUNK
NBUF = 4

SPAN = 640
N_ACC = NS * SPAN

_mesh = plsc.VectorSubcoreMesh(core_axis_name="c", subcore_axis_name="s")


@functools.partial(
    pl.kernel,
    out_type=jax.ShapeDtypeStruct((NC * N_ACC,), jnp.float32),
    mesh=_mesh,
    compiler_params=pltpu.CompilerParams(use_tc_tiling_on_sc=False),
    scratch_types=[
        pltpu.VMEM((EW,), jnp.int32),
        pltpu.VMEM((CHUNK,), jnp.float32),
        pltpu.VMEM((SPAN,), jnp.float32),
        pltpu.VMEM_SHARED((N_ACC,), jnp.float32),
    ],
)
def _sc_degree(ei_hbm, out_hbm, col_v, ones_v, zbuf_v, hist_sh):
    cid = lax.axis_index("c")
    sid = lax.axis_index("s")
    wid = sid * NC + cid
    ebase = wid * EW

    def _fill(i, _):
        zbuf_v[pl.ds(i * 16, 16)] = jnp.zeros((16,), jnp.float32)
        return 0

    lax.fori_loop(0, SPAN // 16, _fill, 0)
    for i in range(CHUNK // 16):
        ones_v[pl.ds(i * 16, 16)] = jnp.ones((16,), jnp.float32)

    pltpu.sync_copy(zbuf_v, hist_sh.at[pl.ds(sid * SPAN, SPAN)])
    plsc.subcore_barrier()

    pltpu.sync_copy(ei_hbm.at[1, pl.ds(ebase, JD * CHUNK)],
                    col_v.at[pl.ds(0, JD * CHUNK)])
    pltpu.sync_copy(ei_hbm.at[1, pl.ds(ebase + JD * CHUNK, TW)],
                    col_v.at[pl.ds(JD * CHUNK, TW)])

    def _scat(j, _):
        pltpu.sync_copy(ones_v, hist_sh.at[col_v.at[pl.ds(j * CHUNK, CHUNK)]],
                        add=True)
        return 0

    lax.fori_loop(0, JD, _scat, 0)
    pltpu.sync_copy(ones_v.at[pl.ds(0, TW)],
                    hist_sh.at[col_v.at[pl.ds(JD * CHUNK, TW)]], add=True)
    plsc.subcore_barrier()

    pltpu.sync_copy(hist_sh.at[pl.ds(sid * SPAN, SPAN)], zbuf_v)
    pltpu.sync_copy(zbuf_v, out_hbm.at[pl.ds(cid * N_ACC + sid * SPAN, SPAN)])


@functools.partial(
    pl.kernel,
    out_type=jax.ShapeDtypeStruct((NC, N_ACC, HD), jnp.float32),
    mesh=_mesh,
    compiler_params=pltpu.CompilerParams(use_tc_tiling_on_sc=False),
    scratch_types=[
        pltpu.VMEM((ET,), jnp.int32),
        pltpu.VMEM((ET,), jnp.int32),
        [pltpu.VMEM((CHUNK, HD), jnp.float32) for _ in range(NBUF)],
        [pltpu.SemaphoreType.DMA for _ in range(NBUF)],
        pltpu.VMEM_SHARED((N_ACC, HD), jnp.float32),
    ],
)
def _sc_aggregate(gl_hbm, gr_hbm, ei_hbm, out_hbm,
                  row_v, col_v, bufs, sems, agg_sh):
    cid = lax.axis_index("c")
    sid = lax.axis_index("s")
    base = sid * SPAN
    ebase = sid * ET

    def _zrow(r, _):
        for i in range(HD // 16):
            bufs[0][r, pl.ds(i * 16, 16)] = jnp.zeros((16,), jnp.float32)
        return 0

    lax.fori_loop(0, CHUNK, _zrow, 0)
    for k in range(SPAN // CHUNK):
        pltpu.sync_copy(bufs[0], agg_sh.at[pl.ds(base + k * CHUNK, CHUNK), :])
    plsc.subcore_barrier()

    pltpu.sync_copy(ei_hbm.at[0, pl.ds(ebase, JA * CHUNK)],
                    row_v.at[pl.ds(0, JA * CHUNK)])
    pltpu.sync_copy(ei_hbm.at[0, pl.ds(ebase + JA * CHUNK, TA)],
                    row_v.at[pl.ds(JA * CHUNK, TA)])
    pltpu.sync_copy(ei_hbm.at[1, pl.ds(ebase, JA * CHUNK)],
                    col_v.at[pl.ds(0, JA * CHUNK)])
    pltpu.sync_copy(ei_hbm.at[1, pl.ds(ebase + JA * CHUNK, TA)],
                    col_v.at[pl.ds(JA * CHUNK, TA)])

    def _run(g_hbm):
        for t in range(NBUF):
            pltpu.async_copy(g_hbm.at[row_v.at[pl.ds(t * CHUNK, CHUNK)]],
                             bufs[t], sems[t])

        def _step(i, _):
            for t in range(NBUF):
                j = i * NBUF + t
                pltpu.make_async_copy(
                    g_hbm.at[row_v.at[pl.ds(0, CHUNK)]], bufs[t],
                    sems[t]).wait()
                pltpu.sync_copy(bufs[t],
                                agg_sh.at[col_v.at[pl.ds(j * CHUNK, CHUNK)]],
                                add=True)

                @pl.when(j + NBUF < JA)
                def _():
                    pltpu.async_copy(
                        g_hbm.at[row_v.at[pl.ds((j + NBUF) * CHUNK, CHUNK)]],
                        bufs[t], sems[t])
            return 0

        lax.fori_loop(0, JA // NBUF, _step, 0)

        pltpu.sync_copy(g_hbm.at[row_v.at[pl.ds(JA * CHUNK, TA)]],
                        bufs[0].at[pl.ds(0, TA), :])
        pltpu.sync_copy(bufs[0].at[pl.ds(0, TA), :],
                        agg_sh.at[col_v.at[pl.ds(JA * CHUNK, TA)]], add=True)

    @pl.when(cid == 0)
    def _():
        _run(gl_hbm)

    @pl.when(cid == 1)
    def _():
        _run(gr_hbm)

    plsc.subcore_barrier()
    for k in range(SPAN // CHUNK):
        t = k % NBUF
        pltpu.sync_copy(agg_sh.at[pl.ds(base + k * CHUNK, CHUNK), :], bufs[t])
        pltpu.sync_copy(bufs[t], out_hbm.at[cid, pl.ds(base + k * CHUNK,
                                                       CHUNK), :])


def _tc_matmul_body(x_ref, w_ref, h_ref):
    h_ref[...] = jnp.dot(x_ref[...], w_ref[...],
                         preferred_element_type=jnp.float32)


def _tc_transform_body(h_ref, h0_ref, h1_ref, gl_ref, gr_ref, dis_ref):
    deg = h0_ref[...] + h1_ref[...] + 1.0
    dis = lax.rsqrt(deg)
    g = h_ref[...] * dis
    gl_ref[...] = g[:, :HD]
    gr_ref[...] = g[:, HD:]
    dis_ref[...] = dis


def _tc_bn_body(a0_ref, a1_ref, gl_ref, gr_ref, dis_ref, b_ref, gam_ref,
                bet_ref, out_ref):
    dis = dis_ref[...]
    agg = jnp.concatenate([a0_ref[...] + gl_ref[...],
                           a1_ref[...] + gr_ref[...]], axis=1)
    pre = dis * agg + b_ref[...]
    mean = jnp.mean(pre, axis=0, keepdims=True)
    d = pre - mean
    var = jnp.mean(d * d, axis=0, keepdims=True)
    out = d * lax.rsqrt(var + 1e-5) * gam_ref[...] + bet_ref[...]
    out_ref[...] = jnp.maximum(out, 0.0)


def kernel(x, edge_index, W, b, gamma, beta):
    hist = _sc_degree(edge_index)
    h0 = hist[:N].reshape(N, 1)
    h1 = hist[N_ACC:N_ACC + N].reshape(N, 1)

    h = pl.pallas_call(
        _tc_matmul_body,
        out_shape=jax.ShapeDtypeStruct((N, D), jnp.float32),
    )(x, W)

    gl, gr, dis = pl.pallas_call(
        _tc_transform_body,
        out_shape=(
            jax.ShapeDtypeStruct((N, HD), jnp.float32),
            jax.ShapeDtypeStruct((N, HD), jnp.float32),
            jax.ShapeDtypeStruct((N, 1), jnp.float32),
        ),
    )(h, h0, h1)

    agg = _sc_aggregate(gl, gr, edge_index)
    a0 = agg[0, :N]
    a1 = agg[1, :N]

    out = pl.pallas_call(
        _tc_bn_body,
        out_shape=jax.ShapeDtypeStruct((N, D), jnp.float32),
    )(a0, a1, gl, gr, dis, b.reshape(1, D), gamma.reshape(1, D),
      beta.reshape(1, D))
    return out


# speedup vs baseline: 45.8728x; 1.0915x over previous
import functools

import jax
import jax.numpy as jnp
from jax import lax
from jax.experimental import pallas as pl
from jax.experimental.pallas import tpu as pltpu
from jax.experimental.pallas import tpu_sc as plsc

N = 10000
D = 128
HD = D // 2
E = 320000

NC = 2
NS = 16
NW = NC * NS
CHUNK = 128

EW = E // NW
JD = EW // CHUNK
TW = EW - JD * CHUNK

ET = E // NS
JA = ET // CHUNK
TA = ET - JA * CHUNK
NBUF = 4

SPAN = 640
N_ACC = NS * SPAN

_mesh = plsc.VectorSubcoreMesh(core_axis_name="c", subcore_axis_name="s")


@functools.partial(
    pl.kernel,
    out_type=jax.ShapeDtypeStruct((NC * N_ACC,), jnp.float32),
    mesh=_mesh,
    compiler_params=pltpu.CompilerParams(use_tc_tiling_on_sc=False),
    scratch_types=[
        pltpu.VMEM((EW,), jnp.int32),
        pltpu.VMEM((CHUNK,), jnp.float32),
        pltpu.VMEM((SPAN,), jnp.float32),
        pltpu.VMEM_SHARED((N_ACC,), jnp.float32),
    ],
)
def _sc_degree(ei_hbm, out_hbm, col_v, ones_v, zbuf_v, hist_sh):
    cid = lax.axis_index("c")
    sid = lax.axis_index("s")
    wid = sid * NC + cid
    ebase = wid * EW

    def _fill(i, _):
        zbuf_v[pl.ds(i * 16, 16)] = jnp.zeros((16,), jnp.float32)
        return 0

    lax.fori_loop(0, SPAN // 16, _fill, 0)
    for i in range(CHUNK // 16):
        ones_v[pl.ds(i * 16, 16)] = jnp.ones((16,), jnp.float32)

    pltpu.sync_copy(zbuf_v, hist_sh.at[pl.ds(sid * SPAN, SPAN)])
    plsc.subcore_barrier()

    pltpu.sync_copy(ei_hbm.at[1, pl.ds(ebase, JD * CHUNK)],
                    col_v.at[pl.ds(0, JD * CHUNK)])
    pltpu.sync_copy(ei_hbm.at[1, pl.ds(ebase + JD * CHUNK, TW)],
                    col_v.at[pl.ds(JD * CHUNK, TW)])

    def _scat(j, _):
        pltpu.sync_copy(ones_v, hist_sh.at[col_v.at[pl.ds(j * CHUNK, CHUNK)]],
                        add=True)
        return 0

    lax.fori_loop(0, JD, _scat, 0)
    pltpu.sync_copy(ones_v.at[pl.ds(0, TW)],
                    hist_sh.at[col_v.at[pl.ds(JD * CHUNK, TW)]], add=True)
    plsc.subcore_barrier()

    pltpu.sync_copy(hist_sh.at[pl.ds(sid * SPAN, SPAN)], zbuf_v)
    pltpu.sync_copy(zbuf_v, out_hbm.at[pl.ds(cid * N_ACC + sid * SPAN, SPAN)])


@functools.partial(
    pl.kernel,
    out_type=jax.ShapeDtypeStruct((NC, N_ACC, HD), jnp.float32),
    mesh=_mesh,
    compiler_params=pltpu.CompilerParams(use_tc_tiling_on_sc=False),
    scratch_types=[
        pltpu.VMEM((ET,), jnp.int32),
        pltpu.VMEM((ET,), jnp.int32),
        [pltpu.VMEM((CHUNK, HD), jnp.float32) for _ in range(NBUF)],
        [pltpu.SemaphoreType.DMA for _ in range(NBUF)],
        pltpu.VMEM_SHARED((N_ACC, HD), jnp.float32),
    ],
)
def _sc_aggregate(gl_hbm, gr_hbm, ei_hbm, out_hbm,
                  row_v, col_v, bufs, sems, agg_sh):
    cid = lax.axis_index("c")
    sid = lax.axis_index("s")
    base = sid * SPAN
    ebase = sid * ET

    def _zrow(r, _):
        for i in range(HD // 16):
            bufs[0][r, pl.ds(i * 16, 16)] = jnp.zeros((16,), jnp.float32)
        return 0

    lax.fori_loop(0, CHUNK, _zrow, 0)
    for k in range(SPAN // CHUNK):
        pltpu.sync_copy(bufs[0], agg_sh.at[pl.ds(base + k * CHUNK, CHUNK), :])
    plsc.subcore_barrier()

    pltpu.sync_copy(ei_hbm.at[0, pl.ds(ebase, JA * CHUNK)],
                    row_v.at[pl.ds(0, JA * CHUNK)])
    pltpu.sync_copy(ei_hbm.at[0, pl.ds(ebase + JA * CHUNK, TA)],
                    row_v.at[pl.ds(JA * CHUNK, TA)])
    pltpu.sync_copy(ei_hbm.at[1, pl.ds(ebase, JA * CHUNK)],
                    col_v.at[pl.ds(0, JA * CHUNK)])
    pltpu.sync_copy(ei_hbm.at[1, pl.ds(ebase + JA * CHUNK, TA)],
                    col_v.at[pl.ds(JA * CHUNK, TA)])

    def _run(g_hbm):
        for t in range(NBUF):
            pltpu.async_copy(g_hbm.at[row_v.at[pl.ds(t * CHUNK, CHUNK)]],
                             bufs[t], sems[t])

        def _step(i, _):
            for t in range(NBUF):
                j = i * NBUF + t
                pltpu.make_async_copy(
                    g_hbm.at[row_v.at[pl.ds(0, CHUNK)]], bufs[t],
                    sems[t]).wait()
                pltpu.sync_copy(bufs[t],
                                agg_sh.at[col_v.at[pl.ds(j * CHUNK, CHUNK)]],
                                add=True)

                @pl.when(j + NBUF < JA)
                def _():
                    pltpu.async_copy(
                        g_hbm.at[row_v.at[pl.ds((j + NBUF) * CHUNK, CHUNK)]],
                        bufs[t], sems[t])
            return 0

        lax.fori_loop(0, JA // NBUF, _step, 0)

        pltpu.sync_copy(g_hbm.at[row_v.at[pl.ds(JA * CHUNK, TA)]],
                        bufs[0].at[pl.ds(0, TA), :])
        pltpu.sync_copy(bufs[0].at[pl.ds(0, TA), :],
                        agg_sh.at[col_v.at[pl.ds(JA * CHUNK, TA)]], add=True)

    @pl.when(cid == 0)
    def _():
        _run(gl_hbm)

    @pl.when(cid == 1)
    def _():
        _run(gr_hbm)

    plsc.subcore_barrier()
    for k in range(SPAN // CHUNK):
        t = k % NBUF
        pltpu.sync_copy(agg_sh.at[pl.ds(base + k * CHUNK, CHUNK), :], bufs[t])
        pltpu.sync_copy(bufs[t], out_hbm.at[cid, pl.ds(base + k * CHUNK,
                                                       CHUNK), :])


def _tc_matmul_body(x_ref, w_ref, h_ref):
    h_ref[...] = jnp.dot(x_ref[...], w_ref[...],
                         preferred_element_type=jnp.float32)


def _tc_transform_body(h_ref, hs_ref, gl_ref, gr_ref):
    dis = lax.rsqrt(hs_ref[...] + 1.0)
    g = h_ref[...] * dis
    gl_ref[...] = g[:, :HD]
    gr_ref[...] = g[:, HD:]


def _tc_bn_body(agg_ref, gl_ref, gr_ref, hs_ref, b_ref, gam_ref,
                bet_ref, out_ref):
    dis = lax.rsqrt(hs_ref[...] + 1.0)
    agg = jnp.concatenate([agg_ref[0, :N, :] + gl_ref[...],
                           agg_ref[1, :N, :] + gr_ref[...]], axis=1)
    pre = dis * agg + b_ref[...]
    mean = jnp.mean(pre, axis=0, keepdims=True)
    d = pre - mean
    var = jnp.mean(d * d, axis=0, keepdims=True)
    out = d * lax.rsqrt(var + 1e-5) * gam_ref[...] + bet_ref[...]
    out_ref[...] = jnp.maximum(out, 0.0)


def kernel(x, edge_index, W, b, gamma, beta):
    hist = _sc_degree(edge_index)
    hs = (hist[:N] + hist[N_ACC:N_ACC + N]).reshape(N, 1)

    h = pl.pallas_call(
        _tc_matmul_body,
        out_shape=jax.ShapeDtypeStruct((N, D), jnp.float32),
    )(x, W)

    gl, gr = pl.pallas_call(
        _tc_transform_body,
        out_shape=(
            jax.ShapeDtypeStruct((N, HD), jnp.float32),
            jax.ShapeDtypeStruct((N, HD), jnp.float32),
        ),
    )(h, hs)

    agg = _sc_aggregate(gl, gr, edge_index)

    out = pl.pallas_call(
        _tc_bn_body,
        out_shape=jax.ShapeDtypeStruct((N, D), jnp.float32),
    )(agg, gl, gr, hs, b.reshape(1, D), gamma.reshape(1, D),
      beta.reshape(1, D))
    return out


# speedup vs baseline: 46.8257x; 1.0208x over previous
import functools

import jax
import jax.numpy as jnp
from jax import lax
from jax.experimental import pallas as pl
from jax.experimental.pallas import tpu as pltpu
from jax.experimental.pallas import tpu_sc as plsc

N = 10000
D = 128
HD = D // 2
E = 320000

NC = 2
NS = 16
NW = NC * NS
CHUNK = 128

EW = E // NW
SLABD = 512
JD = EW // SLABD
TW = EW - JD * SLABD

ET = E // NS
JA = ET // CHUNK
TA = ET - JA * CHUNK
NBUF = 4

SPAN = 640
N_ACC = NS * SPAN

_mesh = plsc.VectorSubcoreMesh(core_axis_name="c", subcore_axis_name="s")


@functools.partial(
    pl.kernel,
    out_type=jax.ShapeDtypeStruct((NC * N_ACC,), jnp.float32),
    mesh=_mesh,
    compiler_params=pltpu.CompilerParams(use_tc_tiling_on_sc=False),
    scratch_types=[
        pltpu.VMEM((EW,), jnp.int32),
        pltpu.VMEM((SLABD,), jnp.float32),
        pltpu.VMEM((SPAN,), jnp.float32),
        pltpu.VMEM_SHARED((N_ACC,), jnp.float32),
    ],
)
def _sc_degree(ei_hbm, out_hbm, col_v, ones_v, zbuf_v, hist_sh):
    cid = lax.axis_index("c")
    sid = lax.axis_index("s")
    wid = sid * NC + cid
    ebase = wid * EW

    def _fill(i, _):
        zbuf_v[pl.ds(i * 16, 16)] = jnp.zeros((16,), jnp.float32)
        return 0

    lax.fori_loop(0, SPAN // 16, _fill, 0)

    def _fill1(i, _):
        ones_v[pl.ds(i * 16, 16)] = jnp.ones((16,), jnp.float32)
        return 0

    lax.fori_loop(0, SLABD // 16, _fill1, 0)

    pltpu.sync_copy(zbuf_v, hist_sh.at[pl.ds(sid * SPAN, SPAN)])
    plsc.subcore_barrier()

    pltpu.sync_copy(ei_hbm.at[1, pl.ds(ebase, JD * SLABD)],
                    col_v.at[pl.ds(0, JD * SLABD)])
    pltpu.sync_copy(ei_hbm.at[1, pl.ds(ebase + JD * SLABD, TW)],
                    col_v.at[pl.ds(JD * SLABD, TW)])

    def _scat(j, _):
        pltpu.sync_copy(ones_v, hist_sh.at[col_v.at[pl.ds(j * SLABD, SLABD)]],
                        add=True)
        return 0

    lax.fori_loop(0, JD, _scat, 0)
    pltpu.sync_copy(ones_v.at[pl.ds(0, TW)],
                    hist_sh.at[col_v.at[pl.ds(JD * SLABD, TW)]], add=True)
    plsc.subcore_barrier()

    pltpu.sync_copy(hist_sh.at[pl.ds(sid * SPAN, SPAN)], zbuf_v)
    pltpu.sync_copy(zbuf_v, out_hbm.at[pl.ds(cid * N_ACC + sid * SPAN, SPAN)])


@functools.partial(
    pl.kernel,
    out_type=jax.ShapeDtypeStruct((NC, N_ACC, HD), jnp.float32),
    mesh=_mesh,
    compiler_params=pltpu.CompilerParams(use_tc_tiling_on_sc=False),
    scratch_types=[
        pltpu.VMEM((ET,), jnp.int32),
        pltpu.VMEM((ET,), jnp.int32),
        [pltpu.VMEM((CHUNK, HD), jnp.float32) for _ in range(NBUF)],
        [pltpu.SemaphoreType.DMA for _ in range(NBUF)],
        pltpu.VMEM_SHARED((N_ACC, HD), jnp.float32),
    ],
)
def _sc_aggregate(gl_hbm, gr_hbm, ei_hbm, out_hbm,
                  row_v, col_v, bufs, sems, agg_sh):
    cid = lax.axis_index("c")
    sid = lax.axis_index("s")
    base = sid * SPAN
    ebase = sid * ET

    def _zrow(r, _):
        for i in range(HD // 16):
            bufs[0][r, pl.ds(i * 16, 16)] = jnp.zeros((16,), jnp.float32)
        return 0

    lax.fori_loop(0, CHUNK, _zrow, 0)
    for k in range(SPAN // CHUNK):
        pltpu.sync_copy(bufs[0], agg_sh.at[pl.ds(base + k * CHUNK, CHUNK), :])
    plsc.subcore_barrier()

    pltpu.sync_copy(ei_hbm.at[0, pl.ds(ebase, JA * CHUNK)],
                    row_v.at[pl.ds(0, JA * CHUNK)])
    pltpu.sync_copy(ei_hbm.at[0, pl.ds(ebase + JA * CHUNK, TA)],
                    row_v.at[pl.ds(JA * CHUNK, TA)])
    pltpu.sync_copy(ei_hbm.at[1, pl.ds(ebase, JA * CHUNK)],
                    col_v.at[pl.ds(0, JA * CHUNK)])
    pltpu.sync_copy(ei_hbm.at[1, pl.ds(ebase + JA * CHUNK, TA)],
                    col_v.at[pl.ds(JA * CHUNK, TA)])

    def _run(g_hbm):
        for t in range(NBUF):
            pltpu.async_copy(g_hbm.at[row_v.at[pl.ds(t * CHUNK, CHUNK)]],
                             bufs[t], sems[t])

        def _step(i, _):
            for t in range(NBUF):
                j = i * NBUF + t
                pltpu.make_async_copy(
                    g_hbm.at[row_v.at[pl.ds(0, CHUNK)]], bufs[t],
                    sems[t]).wait()
                pltpu.sync_copy(bufs[t],
                                agg_sh.at[col_v.at[pl.ds(j * CHUNK, CHUNK)]],
                                add=True)

                @pl.when(j + NBUF < JA)
                def _():
                    pltpu.async_copy(
                        g_hbm.at[row_v.at[pl.ds((j + NBUF) * CHUNK, CHUNK)]],
                        bufs[t], sems[t])
            return 0

        lax.fori_loop(0, JA // NBUF, _step, 0)

        pltpu.sync_copy(g_hbm.at[row_v.at[pl.ds(JA * CHUNK, TA)]],
                        bufs[0].at[pl.ds(0, TA), :])
        pltpu.sync_copy(bufs[0].at[pl.ds(0, TA), :],
                        agg_sh.at[col_v.at[pl.ds(JA * CHUNK, TA)]], add=True)

    @pl.when(cid == 0)
    def _():
        _run(gl_hbm)

    @pl.when(cid == 1)
    def _():
        _run(gr_hbm)

    plsc.subcore_barrier()
    for k in range(SPAN // CHUNK):
        t = k % NBUF
        pltpu.sync_copy(agg_sh.at[pl.ds(base + k * CHUNK, CHUNK), :], bufs[t])
        pltpu.sync_copy(bufs[t], out_hbm.at[cid, pl.ds(base + k * CHUNK,
                                                       CHUNK), :])


def _tc_matmul_body(x_ref, w_ref, h_ref):
    h_ref[...] = jnp.dot(x_ref[...], w_ref[...],
                         preferred_element_type=jnp.float32)


def _tc_transform_body(h_ref, hs_ref, gl_ref, gr_ref):
    dis = lax.rsqrt(hs_ref[...] + 1.0)
    g = h_ref[...] * dis
    gl_ref[...] = g[:, :HD]
    gr_ref[...] = g[:, HD:]


def _tc_bn_body(agg_ref, gl_ref, gr_ref, hs_ref, b_ref, gam_ref,
                bet_ref, out_ref):
    dis = lax.rsqrt(hs_ref[...] + 1.0)
    agg = jnp.concatenate([agg_ref[0, :N, :] + gl_ref[...],
                           agg_ref[1, :N, :] + gr_ref[...]], axis=1)
    pre = dis * agg + b_ref[...]
    mean = jnp.mean(pre, axis=0, keepdims=True)
    d = pre - mean
    var = jnp.mean(d * d, axis=0, keepdims=True)
    out = d * lax.rsqrt(var + 1e-5) * gam_ref[...] + bet_ref[...]
    out_ref[...] = jnp.maximum(out, 0.0)


def kernel(x, edge_index, W, b, gamma, beta):
    hist = _sc_degree(edge_index)
    hs = (hist[:N] + hist[N_ACC:N_ACC + N]).reshape(N, 1)

    h = pl.pallas_call(
        _tc_matmul_body,
        out_shape=jax.ShapeDtypeStruct((N, D), jnp.float32),
    )(x, W)

    gl, gr = pl.pallas_call(
        _tc_transform_body,
        out_shape=(
            jax.ShapeDtypeStruct((N, HD), jnp.float32),
            jax.ShapeDtypeStruct((N, HD), jnp.float32),
        ),
    )(h, hs)

    agg = _sc_aggregate(gl, gr, edge_index)

    out = pl.pallas_call(
        _tc_bn_body,
        out_shape=jax.ShapeDtypeStruct((N, D), jnp.float32),
    )(agg, gl, gr, hs, b.reshape(1, D), gamma.reshape(1, D),
      beta.reshape(1, D))
    return out
